# baseline (speedup 1.0000x reference)
import functools

import jax
import jax.numpy as jnp
from jax import lax
from jax.experimental import pallas as pl
from jax.experimental.pallas import tpu as pltpu
from jax.experimental.pallas import tpu_sc as plsc

B, S = 16384, 200
TAB = 1000000
NC, NS = 2, 16
NW = NC * NS
ROWS_W = B // NW
RCHUNK = 32
RCHUNKS = ROWS_W // RCHUNK
RSTRIDE = 208
FLAT = RCHUNK * RSTRIDE
MCOLS = tuple(range(0, 192, 16)) + (184,)
NHB = 1024
NSP = FLAT - NHB
PIECE = 8000
PIECES = TAB // PIECE

_mesh = plsc.VectorSubcoreMesh(core_axis_name="c", subcore_axis_name="s")


@functools.partial(
    pl.kernel,
    mesh=_mesh,
    out_type=jax.ShapeDtypeStruct((B, S), jnp.float32),
    scratch_types=[
        pltpu.VMEM_SHARED((TAB,), jnp.float32),
        pltpu.VMEM((PIECE,), jnp.float32),
        pltpu.VMEM((RCHUNK, S), jnp.int32),
        pltpu.VMEM((RCHUNK, S), jnp.int32),
        pltpu.VMEM((RCHUNK, S), jnp.float32),
        pltpu.VMEM((RCHUNK, S), jnp.float32),
        pltpu.VMEM((FLAT,), jnp.int32),
        pltpu.VMEM((FLAT,), jnp.int32),
        pltpu.VMEM((FLAT,), jnp.float32),
        pltpu.VMEM((FLAT,), jnp.float32),
        pltpu.SemaphoreType.DMA,
        pltpu.SemaphoreType.DMA,
        pltpu.SemaphoreType.DMA,
        pltpu.SemaphoreType.DMA,
        pltpu.SemaphoreType.DMA,
        pltpu.SemaphoreType.DMA,
        pltpu.SemaphoreType.DMA,
        pltpu.SemaphoreType.DMA,
    ],
)
def _gather_sc(idx_hbm, u_hbm, out_hbm, u_sp, bounce,
               ti0, ti1, to0, to1, if0, if1, of0, of1,
               sin0, sin1, sg0, sg1, sh0, sh1, sout0, sout1):
    sid = lax.axis_index("s")
    wid = sid * NC + lax.axis_index("c")
    tmp_i, tmp_o = (ti0, ti1), (to0, to1)
    idx_f, out_f = (if0, if1), (of0, of1)
    sin, sg, sout = (sin0, sin1), (sg0, sg1), (sout0, sout1)
    sh = (sh0, sh1)
    r00 = wid * ROWS_W

    def in_cp(k, b):
        return pltpu.make_async_copy(
            idx_hbm.at[pl.ds(r00 + k * RCHUNK, RCHUNK), :], tmp_i[b], sin[b])

    def out_cp(k, b):
        return pltpu.make_async_copy(
            tmp_o[b], out_hbm.at[pl.ds(r00 + k * RCHUNK, RCHUNK), :], sout[b])

    def gather_cp(b):
        return pltpu.make_async_copy(
            u_sp.at[idx_f[b].at[pl.ds(0, NSP)]],
            out_f[b].at[pl.ds(0, NSP)], sg[b])

    def gather_hbm_cp(b):
        return pltpu.make_async_copy(
            u_hbm.at[idx_f[b].at[pl.ds(NSP, NHB)]],
            out_f[b].at[pl.ds(NSP, NHB)], sh[b])

    def pack(b):
        for r in range(RCHUNK):
            for c in MCOLS:
                idx_f[b][pl.ds(r * RSTRIDE + c, 16)] = tmp_i[b][r, pl.ds(c, 16)]

    def unpack(b):
        for r in range(RCHUNK):
            for c in MCOLS:
                tmp_o[b][r, pl.ds(c, 16)] = out_f[b][pl.ds(r * RSTRIDE + c, 16)]

    for b in (0, 1):
        for r in range(RCHUNK):
            base = lax.iota(jnp.int32, 16) * 8 + (r * 64 + b * 2048)
            idx_f[b][pl.ds(r * RSTRIDE + 192, 16)] = base

    for j in range((PIECES + NS - 1) // NS):
        piece = sid + NS * j

        @pl.when(piece < PIECES)
        def _stage():
            off = piece * PIECE
            pltpu.sync_copy(u_hbm.at[pl.ds(off, PIECE)], bounce)
            pltpu.sync_copy(bounce, u_sp.at[pl.ds(off, PIECE)])

    plsc.subcore_barrier()

    in_cp(0, 0).start()

    @pl.loop(0, RCHUNKS // 2)
    def _pair(j):
        for half in (0, 1):
            k = 2 * j + half
            b = half
            in_cp(k, b).wait()
            pack(b)
            gather_hbm_cp(b).start()
            gather_cp(b).start()

            @pl.when(k + 1 < RCHUNKS)
            def _prefetch():
                in_cp(k + 1, 1 - b).start()

            @pl.when(k >= 1)
            def _phase2():
                bb = 1 - b
                gather_cp(bb).wait()
                gather_hbm_cp(bb).wait()

                @pl.when(k >= 3)
                def _drain_out():
                    out_cp(k - 3, bb).wait()

                unpack(bb)
                out_cp(k - 1, bb).start()

    gather_cp(1).wait()
    gather_hbm_cp(1).wait()
    out_cp(RCHUNKS - 3, 1).wait()
    unpack(1)
    out_cp(RCHUNKS - 1, 1).start()
    out_cp(RCHUNKS - 2, 0).wait()
    out_cp(RCHUNKS - 1, 1).wait()


def kernel(idx, u):
    return _gather_sc(idx, u)


# speedup vs baseline: 1.1399x; 1.1399x over previous
import functools

import jax
import jax.numpy as jnp
from jax import lax
from jax.experimental import pallas as pl
from jax.experimental.pallas import tpu as pltpu
from jax.experimental.pallas import tpu_sc as plsc

B, S = 16384, 200
TAB = 1000000
NC, NS = 2, 16
NW = NC * NS
ROWS_W = B // NW
RCHUNK = 32
RCHUNKS = ROWS_W // RCHUNK
RSTRIDE = 208
FLAT = RCHUNK * RSTRIDE
MCOLS = tuple(range(0, 192, 16)) + (184,)
PIECE = 8000
PIECES = TAB // PIECE

_mesh = plsc.VectorSubcoreMesh(core_axis_name="c", subcore_axis_name="s")


@functools.partial(
    pl.kernel,
    mesh=_mesh,
    out_type=jax.ShapeDtypeStruct((B, S), jnp.float32),
    scratch_types=[
        pltpu.VMEM_SHARED((TAB,), jnp.float32),
        pltpu.VMEM((PIECE,), jnp.float32),
        pltpu.VMEM((RCHUNK, S), jnp.int32),
        pltpu.VMEM((RCHUNK, S), jnp.int32),
        pltpu.VMEM((RCHUNK, S), jnp.float32),
        pltpu.VMEM((RCHUNK, S), jnp.float32),
        pltpu.VMEM((FLAT,), jnp.int32),
        pltpu.VMEM((FLAT,), jnp.int32),
        pltpu.VMEM((FLAT,), jnp.float32),
        pltpu.VMEM((FLAT,), jnp.float32),
        pltpu.SemaphoreType.DMA,
        pltpu.SemaphoreType.DMA,
        pltpu.SemaphoreType.DMA,
        pltpu.SemaphoreType.DMA,
        pltpu.SemaphoreType.DMA,
        pltpu.SemaphoreType.DMA,
    ],
)
def _gather_sc(idx_hbm, u_hbm, out_hbm, u_sp, bounce,
               ti0, ti1, to0, to1, if0, if1, of0, of1,
               sin0, sin1, sg0, sg1, sout0, sout1):
    sid = lax.axis_index("s")
    wid = sid * NC + lax.axis_index("c")
    tmp_i, tmp_o = (ti0, ti1), (to0, to1)
    idx_f, out_f = (if0, if1), (of0, of1)
    sin, sg, sout = (sin0, sin1), (sg0, sg1), (sout0, sout1)
    r00 = wid * ROWS_W

    def in_cp(k, b):
        return pltpu.make_async_copy(
            idx_hbm.at[pl.ds(r00 + k * RCHUNK, RCHUNK), :], tmp_i[b], sin[b])

    def out_cp(k, b):
        return pltpu.make_async_copy(
            tmp_o[b], out_hbm.at[pl.ds(r00 + k * RCHUNK, RCHUNK), :], sout[b])

    def gather_cp(b):
        return pltpu.make_async_copy(u_sp.at[idx_f[b]], out_f[b], sg[b])

    def pack(b):
        for r in range(RCHUNK):
            for c in MCOLS:
                idx_f[b][pl.ds(r * RSTRIDE + c, 16)] = tmp_i[b][r, pl.ds(c, 16)]

    def unpack(b):
        for r in range(RCHUNK):
            for c in MCOLS:
                tmp_o[b][r, pl.ds(c, 16)] = out_f[b][pl.ds(r * RSTRIDE + c, 16)]

    for b in (0, 1):
        for r in range(RCHUNK):
            base = lax.iota(jnp.int32, 16) * 8 + (r * 64 + b * 2048)
            idx_f[b][pl.ds(r * RSTRIDE + 192, 16)] = base

    for j in range((PIECES + NS - 1) // NS):
        piece = sid + NS * j

        @pl.when(piece < PIECES)
        def _stage():
            off = piece * PIECE
            pltpu.sync_copy(u_hbm.at[pl.ds(off, PIECE)], bounce)
            pltpu.sync_copy(bounce, u_sp.at[pl.ds(off, PIECE)])

    plsc.subcore_barrier()

    in_cp(0, 0).start()

    @pl.loop(0, RCHUNKS // 2)
    def _pair(j):
        for half in (0, 1):
            k = 2 * j + half
            b = half
            in_cp(k, b).wait()
            pack(b)
            gather_cp(b).start()

            @pl.when(k + 1 < RCHUNKS)
            def _prefetch():
                in_cp(k + 1, 1 - b).start()

            @pl.when(k >= 1)
            def _phase2():
                bb = 1 - b
                gather_cp(bb).wait()

                @pl.when(k >= 3)
                def _drain_out():
                    out_cp(k - 3, bb).wait()

                unpack(bb)
                out_cp(k - 1, bb).start()

    gather_cp(1).wait()
    out_cp(RCHUNKS - 3, 1).wait()
    unpack(1)
    out_cp(RCHUNKS - 1, 1).start()
    out_cp(RCHUNKS - 2, 0).wait()
    out_cp(RCHUNKS - 1, 1).wait()


def kernel(idx, u):
    return _gather_sc(idx, u)


# speedup vs baseline: 1.1452x; 1.0046x over previous
import functools

import jax
import jax.numpy as jnp
from jax import lax
from jax.experimental import pallas as pl
from jax.experimental.pallas import tpu as pltpu
from jax.experimental.pallas import tpu_sc as plsc

B, S = 16384, 200
TAB = 1000000
NC, NS = 2, 16
NW = NC * NS
ROWS_W = B // NW
RCHUNK = 32
RCHUNKS = ROWS_W // RCHUNK
RSTRIDE = 208
FLAT = RCHUNK * RSTRIDE
MCOLS = tuple(range(0, 192, 16)) + (184,)
PIECE = 8000
PIECES = TAB // PIECE

_mesh = plsc.VectorSubcoreMesh(core_axis_name="c", subcore_axis_name="s")


@functools.partial(
    pl.kernel,
    mesh=_mesh,
    out_type=jax.ShapeDtypeStruct((B, S), jnp.float32),
    scratch_types=[
        pltpu.VMEM_SHARED((TAB,), jnp.float32),
        pltpu.VMEM((PIECE,), jnp.float32),
        pltpu.VMEM((RCHUNK, S), jnp.int32),
        pltpu.VMEM((RCHUNK, S), jnp.int32),
        pltpu.VMEM((RCHUNK, S), jnp.float32),
        pltpu.VMEM((RCHUNK, S), jnp.float32),
        pltpu.VMEM((FLAT,), jnp.int32),
        pltpu.VMEM((FLAT,), jnp.int32),
        pltpu.VMEM((FLAT,), jnp.float32),
        pltpu.VMEM((FLAT,), jnp.float32),
        pltpu.SemaphoreType.DMA,
        pltpu.SemaphoreType.DMA,
        pltpu.SemaphoreType.DMA,
        pltpu.SemaphoreType.DMA,
        pltpu.SemaphoreType.DMA,
        pltpu.SemaphoreType.DMA,
    ],
)
def _gather_sc(idx_hbm, u_hbm, out_hbm, u_sp, bounce,
               ti0, ti1, to0, to1, if0, if1, of0, of1,
               sin0, sin1, sg0, sg1, sout0, sout1):
    sid = lax.axis_index("s")
    wid = sid * NC + lax.axis_index("c")
    tmp_i, tmp_o = (ti0, ti1), (to0, to1)
    idx_f, out_f = (if0, if1), (of0, of1)
    sin, sg, sout = (sin0, sin1), (sg0, sg1), (sout0, sout1)
    r00 = wid * ROWS_W

    def in_cp(k, b):
        return pltpu.make_async_copy(
            idx_hbm.at[pl.ds(r00 + k * RCHUNK, RCHUNK), :], tmp_i[b], sin[b])

    def out_cp(k, b):
        return pltpu.make_async_copy(
            tmp_o[b], out_hbm.at[pl.ds(r00 + k * RCHUNK, RCHUNK), :], sout[b])

    def gather_cp(b):
        return pltpu.make_async_copy(u_sp.at[idx_f[b]], out_f[b], sg[b])

    def pack(b):
        for r in range(RCHUNK):
            for c in MCOLS:
                idx_f[b][pl.ds(r * RSTRIDE + c, 16)] = tmp_i[b][r, pl.ds(c, 16)]

    def unpack(b):
        for r in range(RCHUNK):
            for c in MCOLS:
                tmp_o[b][r, pl.ds(c, 16)] = out_f[b][pl.ds(r * RSTRIDE + c, 16)]

    for b in (0, 1):
        for r in range(RCHUNK):
            base = lax.iota(jnp.int32, 16) * 8 + (r * 64 + b * 2048)
            idx_f[b][pl.ds(r * RSTRIDE + 192, 16)] = base

    in_cp(0, 0).start()

    for j in range((PIECES + NS - 1) // NS):
        piece = sid + NS * j

        @pl.when(piece < PIECES)
        def _stage():
            off = piece * PIECE
            pltpu.sync_copy(u_hbm.at[pl.ds(off, PIECE)], bounce)
            pltpu.sync_copy(bounce, u_sp.at[pl.ds(off, PIECE)])

    in_cp(0, 0).wait()
    pack(0)
    plsc.subcore_barrier()

    @pl.loop(0, RCHUNKS // 2)
    def _pair(j):
        for half in (0, 1):
            k = 2 * j + half
            b = half
            if half == 0:
                @pl.when(j >= 1)
                def _p1():
                    in_cp(k, b).wait()
                    pack(b)
            else:
                in_cp(k, b).wait()
                pack(b)
            gather_cp(b).start()

            @pl.when(k + 1 < RCHUNKS)
            def _prefetch():
                in_cp(k + 1, 1 - b).start()

            @pl.when(k >= 1)
            def _phase2():
                bb = 1 - b
                gather_cp(bb).wait()

                @pl.when(k >= 3)
                def _drain_out():
                    out_cp(k - 3, bb).wait()

                unpack(bb)
                out_cp(k - 1, bb).start()

    gather_cp(1).wait()
    out_cp(RCHUNKS - 3, 1).wait()
    unpack(1)
    out_cp(RCHUNKS - 1, 1).start()
    out_cp(RCHUNKS - 2, 0).wait()
    out_cp(RCHUNKS - 1, 1).wait()


def kernel(idx, u):
    return _gather_sc(idx, u)


# speedup vs baseline: 1.6478x; 1.4389x over previous
import functools

import jax
import jax.numpy as jnp
from jax import lax
from jax.experimental import pallas as pl
from jax.experimental.pallas import tpu as pltpu
from jax.experimental.pallas import tpu_sc as plsc

B, S = 16384, 200
TAB = 1000000
NC, NS = 2, 16
NW = NC * NS
COLS_W = B // NW
JB, IB = 40, 128
NJ = S // JB
NI = COLS_W // IB
NCHUNKS = NJ * NI
PIECE = 8000
PIECES = TAB // PIECE

_mesh = plsc.VectorSubcoreMesh(core_axis_name="c", subcore_axis_name="s")


@functools.partial(
    pl.kernel,
    mesh=_mesh,
    out_type=jax.ShapeDtypeStruct((S, B), jnp.float32),
    scratch_types=[
        pltpu.VMEM_SHARED((TAB,), jnp.float32),
        pltpu.VMEM((PIECE,), jnp.float32),
        pltpu.VMEM((JB, IB), jnp.int32),
        pltpu.VMEM((JB, IB), jnp.int32),
        pltpu.VMEM((JB, IB), jnp.float32),
        pltpu.VMEM((JB, IB), jnp.float32),
        pltpu.SemaphoreType.DMA,
        pltpu.SemaphoreType.DMA,
        pltpu.SemaphoreType.DMA,
        pltpu.SemaphoreType.DMA,
        pltpu.SemaphoreType.DMA,
        pltpu.SemaphoreType.DMA,
    ],
)
def _gather_sc(idx_hbm, u_hbm, out_hbm, u_sp, bounce,
               ti0, ti1, to0, to1, sin0, sin1, sg0, sg1, sout0, sout1):
    sid = lax.axis_index("s")
    wid = sid * NC + lax.axis_index("c")
    tmp_i, tmp_o = (ti0, ti1), (to0, to1)
    sin, sg, sout = (sin0, sin1), (sg0, sg1), (sout0, sout1)
    c00 = wid * COLS_W

    def chunk_slice(k):
        j0 = (k % NJ) * JB
        i0 = c00 + (k // NJ) * IB
        return (pl.ds(j0, JB), pl.ds(i0, IB))

    def in_cp(k, b):
        return pltpu.make_async_copy(idx_hbm.at[chunk_slice(k)], tmp_i[b],
                                     sin[b])

    def out_cp(k, b):
        return pltpu.make_async_copy(tmp_o[b], out_hbm.at[chunk_slice(k)],
                                     sout[b])

    def fire_gathers(b):
        @pl.loop(0, JB)
        def _issue(r):
            pltpu.async_copy(u_sp.at[tmp_i[b].at[r]], tmp_o[b].at[r], sg[b])

    def drain_gathers(b):
        @pl.loop(0, JB)
        def _drain(r):
            pltpu.make_async_copy(u_sp.at[tmp_i[b].at[r]], tmp_o[b].at[r],
                                  sg[b]).wait()

    in_cp(0, 0).start()

    for j in range((PIECES + NS - 1) // NS):
        piece = sid + NS * j

        @pl.when(piece < PIECES)
        def _stage():
            off = piece * PIECE
            pltpu.sync_copy(u_hbm.at[pl.ds(off, PIECE)], bounce)
            pltpu.sync_copy(bounce, u_sp.at[pl.ds(off, PIECE)])

    plsc.subcore_barrier()

    @pl.loop(0, NCHUNKS // 2)
    def _pair(j):
        for half in (0, 1):
            k = 2 * j + half
            b = half
            in_cp(k, b).wait()

            @pl.when(k >= 2)
            def _drain_out_b():
                out_cp(k - 2, b).wait()

            fire_gathers(b)

            @pl.when(k + 1 < NCHUNKS)
            def _prefetch():
                in_cp(k + 1, 1 - b).start()

            @pl.when(k >= 1)
            def _phase2():
                bb = 1 - b
                drain_gathers(bb)
                out_cp(k - 1, bb).start()

    drain_gathers(1)
    out_cp(NCHUNKS - 1, 1).start()
    out_cp(NCHUNKS - 2, 0).wait()
    out_cp(NCHUNKS - 1, 1).wait()


def kernel(idx, u):
    return _gather_sc(idx.T, u).T
